# Initial kernel scaffold; baseline (speedup 1.0000x reference)
import jax
import jax.numpy as jnp
from jax.experimental import pallas as pl


def kernel(logits, candidates):
    raise NotImplementedError("write your pallas kernel here")


# speedup vs baseline: 2.0002x; 2.0002x over previous
import functools

import jax
import jax.numpy as jnp
from jax import lax
from jax.experimental import pallas as pl
from jax.experimental.pallas import tpu as pltpu
from jax.experimental.pallas import tpu_sc as plsc

_HEAD_SIZE = 2000
_TAIL_SAMPLE = 100
_CLIP = 20.0
_LANES = 128


def _psi(u):
    return jnp.log1p(jnp.exp(-u))


@functools.lru_cache(maxsize=None)
def _make_sc_gather(n_rows, n_flat):
    info = plsc.get_sparse_core_info()
    nc, ns = info.num_cores, info.num_subcores
    nw = nc * ns
    rows_per_w = n_rows // nw
    chunk = 128

    mesh = plsc.VectorSubcoreMesh(core_axis_name="c", subcore_axis_name="s")

    @functools.partial(
        pl.kernel,
        mesh=mesh,
        out_type=jax.ShapeDtypeStruct((n_rows, _LANES), jnp.float32),
        scratch_types=[
            pltpu.VMEM((rows_per_w, chunk), jnp.int32),
            pltpu.VMEM((rows_per_w, chunk), jnp.float32),
            pltpu.SemaphoreType.DMA,
        ],
    )
    def sc_gather(flat_hbm, idx_hbm, out_hbm, idx_v, vals_v, sem):
        wid = lax.axis_index("s") * nc + lax.axis_index("c")
        base = wid * rows_per_w
        pltpu.sync_copy(idx_hbm.at[pl.ds(base, rows_per_w), :], idx_v)

        unroll = 8

        def group(g, carry):
            j0 = g * unroll
            copies = [
                pltpu.async_copy(
                    flat_hbm.at[idx_v.at[j0 + u]], vals_v.at[j0 + u], sem)
                for u in range(unroll)
            ]
            for cp in copies:
                cp.wait()
            return carry

        lax.fori_loop(0, rows_per_w // unroll, group, 0)
        pltpu.sync_copy(vals_v, out_hbm.at[pl.ds(base, rows_per_w), :])

    return sc_gather


def _tc_body(nk, ss, hs, tail_scale, inv_b,
             head_ref, gc_ref, gt_ref, c_ref, cols_ref, out_ref):
    x = jnp.clip(head_ref[...], -_CLIP, _CLIP)
    lane = lax.broadcasted_iota(jnp.int32, x.shape, 1)
    s_head = jnp.sum(jnp.where(lane < hs, _psi(-x), 0.0), axis=1,
                     keepdims=True)

    cv = jnp.clip(gc_ref[...], -_CLIP, _CLIP)
    tv = jnp.clip(gt_ref[...], -_CLIP, _CLIP)
    cols = cols_ref[...]

    cjs = [c_ref[:, j:j + 1] for j in range(nk)]
    valids = [cj >= 0 for cj in cjs]
    firsts = []
    for j in range(nk):
        dup = None
        for k in range(j):
            e = cjs[k] == cjs[j]
            dup = e if dup is None else (dup | e)
        f = valids[j] if dup is None else (valids[j] & ~dup)
        firsts.append(f)

    card = None
    s_cand = None
    hcorr = None
    for j in range(nk):
        fj = firsts[j].astype(jnp.float32)
        cvj = cv[:, j:j + 1]
        card = fj if card is None else card + fj
        sj = cvj * fj
        s_cand = sj if s_cand is None else s_cand + sj
        hj = jnp.where(firsts[j] & (cjs[j] < hs), _psi(-cvj), 0.0)
        hcorr = hj if hcorr is None else hcorr + hj
    term1 = _psi(s_cand / jnp.maximum(card, 1.0))
    term2 = s_head - hcorr

    iscand = None
    for j in range(nk):
        e = (cjs[j] == cols) & valids[j]
        iscand = e if iscand is None else (iscand | e)
    t3 = jnp.sum(jnp.where(iscand, 0.0, _psi(-tv)), axis=1,
                 keepdims=True) * tail_scale

    part = jnp.sum(term1 + term2 + t3, axis=0, keepdims=True) * inv_b

    @pl.when(pl.program_id(0) == 0)
    def _init():
        out_ref[...] = jnp.zeros_like(out_ref)

    out_ref[...] += part


def kernel(logits, candidates):
    b, n = logits.shape
    nk = candidates.shape[1]
    hs = min(_HEAD_SIZE, n)
    ts = n - hs
    ss = min(_TAIL_SAMPLE, ts)
    sidx = jax.random.randint(jax.random.key(42), (ss,), 0, ts)
    cols = (hs + sidx).astype(jnp.int32)

    c = candidates.astype(jnp.int32)
    row_base = (jnp.arange(b, dtype=jnp.int32) * n)[:, None]
    idx_cand = row_base + jnp.clip(c, 0, n - 1)
    idx_tail = row_base + cols[None, :]
    npad = _LANES - nk - ss
    idx_pad = jnp.broadcast_to(row_base, (b, npad))
    idx_all = jnp.concatenate([idx_cand, idx_tail, idx_pad], axis=1)

    flat = jnp.reshape(logits, (-1,))
    gathered = _make_sc_gather(b, b * n)(flat, idx_all)
    gc = gathered[:, :nk]
    gt = gathered[:, nk:nk + ss]

    br = 128
    hblk = ((hs + 127) // 128) * 128
    out = pl.pallas_call(
        functools.partial(_tc_body, nk, ss, hs, float(ts) / float(ss), 1.0 / b),
        grid=(b // br,),
        in_specs=[
            pl.BlockSpec((br, hblk), lambda i: (i, 0)),
            pl.BlockSpec((br, nk), lambda i: (i, 0)),
            pl.BlockSpec((br, ss), lambda i: (i, 0)),
            pl.BlockSpec((br, nk), lambda i: (i, 0)),
            pl.BlockSpec((1, ss), lambda i: (0, 0)),
        ],
        out_specs=pl.BlockSpec((1, 1), lambda i: (0, 0)),
        out_shape=jax.ShapeDtypeStruct((1, 1), jnp.float32),
    )(logits, gc, gt, c, cols.reshape(1, ss))
    return out[0, 0]


# speedup vs baseline: 42.5943x; 21.2955x over previous
import functools

import jax
import jax.numpy as jnp
from jax import lax
from jax.experimental import pallas as pl
from jax.experimental.pallas import tpu as pltpu
from jax.experimental.pallas import tpu_sc as plsc

_HEAD_SIZE = 2000
_TAIL_SAMPLE = 100
_CLIP = 20.0
_LW = 128
_NQ = 4
_QROWS = 32
_CCH = 4
_CHW = 80


def _psi(u):
    return jnp.log1p(jnp.exp(-u))


@functools.lru_cache(maxsize=None)
def _make_sc_gather(n_classes, b, nk):
    info = plsc.get_sparse_core_info()
    nc, ns = info.num_cores, info.num_subcores
    nw = nc * ns
    nt = b // _LW
    cand_rows = (b // nt) * nk // _NQ

    mesh = plsc.VectorSubcoreMesh(core_axis_name="c", subcore_axis_name="s")

    @functools.partial(
        pl.kernel,
        mesh=mesh,
        out_type=[
            jax.ShapeDtypeStruct((_NQ * _QROWS, b), jnp.float32),
            jax.ShapeDtypeStruct((b * nk, _LW), jnp.float32),
        ],
        scratch_types=[
            pltpu.VMEM((_QROWS,), jnp.int32),
            pltpu.VMEM((_CCH, _CHW), jnp.int32),
            pltpu.VMEM((_QROWS, _LW), jnp.float32),
            pltpu.VMEM((cand_rows, _LW), jnp.float32),
            pltpu.SemaphoreType.DMA,
        ],
    )
    def sc_gather(lt_hbm, tidx_hbm, cidx_hbm, gt_hbm, gc_hbm,
                  tidx_v, cidx_v, tvals_v, cvals_v, sem):
        wid = lax.axis_index("s") * nc + lax.axis_index("c")
        j = wid // _NQ
        q = wid % _NQ
        col0 = pl.multiple_of(j * _LW, _LW)
        pltpu.sync_copy(tidx_hbm.at[q], tidx_v)
        pltpu.sync_copy(cidx_hbm.at[wid], cidx_v)
        cp_t = pltpu.async_copy(
            lt_hbm.at[tidx_v, pl.ds(col0, _LW)], tvals_v, sem)
        cps = [
            pltpu.async_copy(
                lt_hbm.at[cidx_v.at[t], pl.ds(col0, _LW)],
                cvals_v.at[pl.ds(t * _CHW, _CHW)], sem)
            for t in range(_CCH)
        ]
        cp_t.wait()
        for cp in cps:
            cp.wait()
        pltpu.sync_copy(
            tvals_v, gt_hbm.at[pl.ds(q * _QROWS, _QROWS), pl.ds(col0, _LW)])
        pltpu.sync_copy(
            cvals_v, gc_hbm.at[pl.ds(wid * cand_rows, cand_rows), :])

    return sc_gather


def _tc_body(hs, nk, tail_scale, inv_b,
             lt_ref, gt_ref, gc_ref, cid_ref, cols_ref, out_ref):
    x = jnp.clip(lt_ref[...], -_CLIP, _CLIP)
    row = lax.broadcasted_iota(jnp.int32, x.shape, 0)
    s_head = jnp.sum(jnp.where(row < hs, _psi(-x), 0.0), axis=0,
                     keepdims=True)

    eye = (lax.broadcasted_iota(jnp.int32, (_LW, _LW), 0)
           == lax.broadcasted_iota(jnp.int32, (_LW, _LW), 1))
    cids = [cid_ref[k:k + 1, :] for k in range(nk)]
    cvs = []
    for k in range(nk):
        sub = jnp.clip(gc_ref[k * _LW:(k + 1) * _LW, :], -_CLIP, _CLIP)
        cvs.append(jnp.sum(jnp.where(eye, sub, 0.0), axis=0, keepdims=True))

    valids = [cid >= 0 for cid in cids]
    firsts = []
    for k in range(nk):
        dup = None
        for m in range(k):
            e = cids[m] == cids[k]
            dup = e if dup is None else (dup | e)
        firsts.append(valids[k] if dup is None else (valids[k] & ~dup))

    card = None
    s_cand = None
    hcorr = None
    for k in range(nk):
        fk = firsts[k].astype(jnp.float32)
        card = fk if card is None else card + fk
        sk = cvs[k] * fk
        s_cand = sk if s_cand is None else s_cand + sk
        hk = jnp.where(firsts[k] & (cids[k] < hs), _psi(-cvs[k]), 0.0)
        hcorr = hk if hcorr is None else hcorr + hk
    term1 = _psi(s_cand / jnp.maximum(card, 1.0))
    term2 = s_head - hcorr

    colsb = cols_ref[...]
    row_ok = colsb >= 0
    pt = _psi(-jnp.clip(gt_ref[...], -_CLIP, _CLIP))
    iscand = None
    for k in range(nk):
        e = cids[k] == colsb
        iscand = e if iscand is None else (iscand | e)
    t3 = jnp.sum(jnp.where(row_ok & ~iscand, pt, 0.0), axis=0,
                 keepdims=True) * tail_scale

    part = jnp.sum(term1 + term2 + t3, axis=1, keepdims=True) * inv_b

    @pl.when(pl.program_id(0) == 0)
    def _init():
        out_ref[...] = jnp.zeros_like(out_ref)

    out_ref[...] += part


def kernel(logits, candidates):
    b, n = logits.shape
    nk = candidates.shape[1]
    hs = min(_HEAD_SIZE, n)
    ts = n - hs
    ss = min(_TAIL_SAMPLE, ts)
    ssq = ss // _NQ
    nt = b // _LW
    nw = nt * _NQ

    sidx = jax.random.randint(jax.random.key(42), (ss,), 0, ts)
    cols = (hs + sidx).astype(jnp.int32)

    lt = logits.T
    c_t = candidates.T.astype(jnp.int32)

    tidx = jnp.full((_NQ, _QROWS), cols[0], jnp.int32)
    tidx = tidx.at[:, :ssq].set(cols.reshape(_NQ, ssq))
    c_safe = jnp.clip(c_t, 0, n - 1)
    arr = c_safe.reshape(nk, nt, _LW).transpose(1, 0, 2)
    cidx = arr.reshape(nw, _CCH, _CHW)
    pos = (jnp.arange(ss) // ssq) * _QROWS + jnp.arange(ss) % ssq
    colsarr = jnp.full((_NQ * _QROWS,), -1, jnp.int32).at[pos].set(cols)
    colsb = jnp.broadcast_to(colsarr[:, None], (_NQ * _QROWS, _LW))

    gt, gc = _make_sc_gather(n, b, nk)(lt, tidx, cidx)

    hb = ((hs + _LW - 1) // _LW) * _LW
    out = pl.pallas_call(
        functools.partial(_tc_body, hs, nk, float(ts) / float(ss), 1.0 / b),
        grid=(nt,),
        in_specs=[
            pl.BlockSpec((hb, _LW), lambda i: (0, i)),
            pl.BlockSpec((_NQ * _QROWS, _LW), lambda i: (0, i)),
            pl.BlockSpec((b * nk // nt, _LW), lambda i: (i, 0)),
            pl.BlockSpec((nk, _LW), lambda i: (0, i)),
            pl.BlockSpec((_NQ * _QROWS, _LW), lambda i: (0, 0)),
        ],
        out_specs=pl.BlockSpec((1, 1), lambda i: (0, 0)),
        out_shape=jax.ShapeDtypeStruct((1, 1), jnp.float32),
    )(lt, gt, gc, c_t, colsb)
    return out[0, 0]


# speedup vs baseline: 44.7070x; 1.0496x over previous
import functools

import jax
import jax.numpy as jnp
from jax import lax
from jax.experimental import pallas as pl
from jax.experimental.pallas import tpu as pltpu
from jax.experimental.pallas import tpu_sc as plsc

_HEAD_SIZE = 2000
_TAIL_SAMPLE = 100
_CLIP = 20.0
_LW = 128
_NQ = 4
_QROWS = 32
_CCH = 4
_CHW = 80


def _psi(u):
    return jnp.log1p(jnp.exp(-u))


@functools.lru_cache(maxsize=None)
def _make_sc_gather(n_classes, b, nk):
    info = plsc.get_sparse_core_info()
    nc, ns = info.num_cores, info.num_subcores
    nw = nc * ns
    nt = b // _LW
    cand_rows = (b // nt) * nk // _NQ

    mesh = plsc.VectorSubcoreMesh(core_axis_name="c", subcore_axis_name="s")

    @functools.partial(
        pl.kernel,
        mesh=mesh,
        out_type=[
            jax.ShapeDtypeStruct((_NQ * _QROWS, b), jnp.float32),
            jax.ShapeDtypeStruct((b * nk, _LW), jnp.float32),
        ],
        scratch_types=[
            pltpu.VMEM((_QROWS,), jnp.int32),
            pltpu.VMEM((_CCH, _CHW), jnp.int32),
            pltpu.VMEM((_QROWS, _LW), jnp.float32),
            pltpu.VMEM((cand_rows, _LW), jnp.float32),
            pltpu.SemaphoreType.DMA,
        ],
    )
    def sc_gather(lt_hbm, tidx_hbm, cidx_hbm, gt_hbm, gc_hbm,
                  tidx_v, cidx_v, tvals_v, cvals_v, sem):
        wid = lax.axis_index("s") * nc + lax.axis_index("c")
        j = wid // _NQ
        q = wid % _NQ
        col0 = pl.multiple_of(j * _LW, _LW)
        pltpu.sync_copy(tidx_hbm.at[q], tidx_v)
        pltpu.sync_copy(cidx_hbm.at[wid], cidx_v)
        cp_t = pltpu.async_copy(
            lt_hbm.at[tidx_v, pl.ds(col0, _LW)], tvals_v, sem)
        cps = [
            pltpu.async_copy(
                lt_hbm.at[cidx_v.at[t], pl.ds(col0, _LW)],
                cvals_v.at[pl.ds(t * _CHW, _CHW)], sem)
            for t in range(_CCH)
        ]
        cp_t.wait()
        pltpu.sync_copy(
            tvals_v, gt_hbm.at[pl.ds(q * _QROWS, _QROWS), pl.ds(col0, _LW)])
        for cp in cps:
            cp.wait()
        pltpu.sync_copy(
            cvals_v, gc_hbm.at[pl.ds(wid * cand_rows, cand_rows), :])

    return sc_gather


def _head_body(hs, lt_ref, out_ref):
    x = jnp.clip(lt_ref[...], -_CLIP, _CLIP)
    out_ref[...] = jnp.sum(_psi(-x), axis=0, keepdims=True)


def _combine_body(nk, nt, hs, tail_scale, inv_b,
                  hsum_ref, gt_ref, gc_ref, cid_ref, cols_ref, out_ref):
    cids = [cid_ref[k:k + 1, :] for k in range(nk)]
    eye = (lax.broadcasted_iota(jnp.int32, (_LW, _LW), 0)
           == lax.broadcasted_iota(jnp.int32, (_LW, _LW), 1))
    cvs = []
    for k in range(nk):
        parts = []
        for j in range(nt):
            base = (j * nk + k) * _LW
            sub = gc_ref[base:base + _LW, :]
            parts.append(jnp.sum(jnp.where(eye, sub, 0.0), axis=0,
                                 keepdims=True))
        cvs.append(jnp.clip(jnp.concatenate(parts, axis=1), -_CLIP, _CLIP))

    valids = [cid >= 0 for cid in cids]
    firsts = []
    for k in range(nk):
        dup = None
        for m in range(k):
            e = cids[m] == cids[k]
            dup = e if dup is None else (dup | e)
        firsts.append(valids[k] if dup is None else (valids[k] & ~dup))

    card = None
    s_cand = None
    hcorr = None
    for k in range(nk):
        fk = firsts[k].astype(jnp.float32)
        card = fk if card is None else card + fk
        sk = cvs[k] * fk
        s_cand = sk if s_cand is None else s_cand + sk
        hk = jnp.where(firsts[k] & (cids[k] < hs), _psi(-cvs[k]), 0.0)
        hcorr = hk if hcorr is None else hcorr + hk
    term1 = _psi(s_cand / jnp.maximum(card, 1.0))
    term2 = hsum_ref[...] - hcorr

    colsb = cols_ref[...]
    row_ok = colsb >= 0
    pt = _psi(-jnp.clip(gt_ref[...], -_CLIP, _CLIP))
    iscand = None
    for k in range(nk):
        e = cids[k] == colsb
        iscand = e if iscand is None else (iscand | e)
    t3 = jnp.sum(jnp.where(row_ok & ~iscand, pt, 0.0), axis=0,
                 keepdims=True) * tail_scale

    out_ref[...] = jnp.sum(term1 + term2 + t3, axis=1, keepdims=True) * inv_b


def kernel(logits, candidates):
    b, n = logits.shape
    nk = candidates.shape[1]
    hs = min(_HEAD_SIZE, n)
    ts = n - hs
    ss = min(_TAIL_SAMPLE, ts)
    ssq = ss // _NQ
    nt = b // _LW
    nw = nt * _NQ
    qr = _NQ * _QROWS

    sidx = jax.random.randint(jax.random.key(42), (ss,), 0, ts)
    cols = (hs + sidx).astype(jnp.int32)

    lt = logits.T
    c_t = candidates.T.astype(jnp.int32)

    tidx = jnp.full((_NQ, _QROWS), cols[0], jnp.int32)
    tidx = tidx.at[:, :ssq].set(cols.reshape(_NQ, ssq))
    c_safe = jnp.clip(c_t, 0, n - 1)
    arr = c_safe.reshape(nk, nt, _LW).transpose(1, 0, 2)
    cidx = arr.reshape(nw, _CCH, _CHW)
    pos = (jnp.arange(ss) // ssq) * _QROWS + jnp.arange(ss) % ssq
    colsarr = jnp.full((qr,), -1, jnp.int32).at[pos].set(cols)
    colsb = jnp.broadcast_to(colsarr[:, None], (qr, b))

    gt, gc = _make_sc_gather(n, b, nk)(lt, tidx, cidx)

    hsum = pl.pallas_call(
        functools.partial(_head_body, hs),
        grid=(nt,),
        in_specs=[pl.BlockSpec((hs, _LW), lambda i: (0, i))],
        out_specs=pl.BlockSpec((1, _LW), lambda i: (0, i)),
        out_shape=jax.ShapeDtypeStruct((1, b), jnp.float32),
    )(lt)

    out = pl.pallas_call(
        functools.partial(
            _combine_body, nk, nt, hs, float(ts) / float(ss), 1.0 / b),
        in_specs=[
            pl.BlockSpec((1, b), lambda: (0, 0)),
            pl.BlockSpec((qr, b), lambda: (0, 0)),
            pl.BlockSpec((b * nk, _LW), lambda: (0, 0)),
            pl.BlockSpec((nk, b), lambda: (0, 0)),
            pl.BlockSpec((qr, b), lambda: (0, 0)),
        ],
        out_specs=pl.BlockSpec((1, 1), lambda: (0, 0)),
        out_shape=jax.ShapeDtypeStruct((1, 1), jnp.float32),
    )(hsum, gt, gc, c_t, colsb)
    return out[0, 0]


# speedup vs baseline: 54.0860x; 1.2098x over previous
import functools

import numpy as np

import jax
import jax.numpy as jnp
from jax import lax
from jax.experimental import pallas as pl
from jax.experimental.pallas import tpu as pltpu
from jax.experimental.pallas import tpu_sc as plsc

_HEAD_SIZE = 2000
_TAIL_SAMPLE = 100
_CLIP = 20.0
_LW = 128
_NQ = 4
_QROWS = 32
_CCH = 4
_CHW = 80


def _psi_neg(x):
    return jnp.log1p(jnp.exp(jnp.minimum(x, _CLIP)))


@functools.lru_cache(maxsize=None)
def _make_sc_gather(n_classes, b, nk):
    info = plsc.get_sparse_core_info()
    nc, ns = info.num_cores, info.num_subcores
    nw = nc * ns
    nt = b // _LW
    cand_rows = (b // nt) * nk // _NQ

    mesh = plsc.VectorSubcoreMesh(core_axis_name="c", subcore_axis_name="s")

    @functools.partial(
        pl.kernel,
        mesh=mesh,
        out_type=[
            jax.ShapeDtypeStruct((_NQ * _QROWS, b), jnp.float32),
            jax.ShapeDtypeStruct((b * nk, _LW), jnp.float32),
        ],
        scratch_types=[
            pltpu.VMEM((_QROWS,), jnp.int32),
            pltpu.VMEM((_CCH, _CHW), jnp.int32),
            pltpu.VMEM((_QROWS, _LW), jnp.float32),
            pltpu.VMEM((cand_rows, _LW), jnp.float32),
            pltpu.SemaphoreType.DMA,
        ],
    )
    def sc_gather(lt_hbm, tidx_hbm, cidx_hbm, gt_hbm, gc_hbm,
                  tidx_v, cidx_v, tvals_v, cvals_v, sem):
        wid = lax.axis_index("s") * nc + lax.axis_index("c")
        j = wid // _NQ
        q = wid % _NQ
        col0 = pl.multiple_of(j * _LW, _LW)
        pltpu.sync_copy(tidx_hbm.at[q], tidx_v)
        pltpu.sync_copy(cidx_hbm.at[wid], cidx_v)
        cp_t = pltpu.async_copy(
            lt_hbm.at[tidx_v, pl.ds(col0, _LW)], tvals_v, sem)
        cps = [
            pltpu.async_copy(
                lt_hbm.at[cidx_v.at[t], pl.ds(col0, _LW)],
                cvals_v.at[pl.ds(t * _CHW, _CHW)], sem)
            for t in range(_CCH)
        ]
        cp_t.wait()
        pltpu.sync_copy(
            tvals_v, gt_hbm.at[pl.ds(q * _QROWS, _QROWS), pl.ds(col0, _LW)])
        for cp in cps:
            cp.wait()
        pltpu.sync_copy(
            cvals_v, gc_hbm.at[pl.ds(wid * cand_rows, cand_rows), :])

    return sc_gather


def _head_body(lt_ref, out_ref):
    out_ref[...] = jnp.sum(_psi_neg(lt_ref[...]), axis=0, keepdims=True)


def _combine_body(nk, hs, tail_scale, inv_b,
                  hsum_ref, gt_ref, gc_ref, cid_ref, cols_ref, out_ref):
    cids = [cid_ref[k:k + 1, :] for k in range(nk)]
    eye = (lax.broadcasted_iota(jnp.int32, (_LW, _LW), 0)
           == lax.broadcasted_iota(jnp.int32, (_LW, _LW), 1))
    cvs = []
    for k in range(nk):
        sub = gc_ref[k * _LW:(k + 1) * _LW, :]
        v = jnp.sum(jnp.where(eye, sub, 0.0), axis=0, keepdims=True)
        cvs.append(jnp.clip(v, -_CLIP, _CLIP))

    valids = [cid >= 0 for cid in cids]
    firsts = []
    for k in range(nk):
        dup = None
        for m in range(k):
            e = cids[m] == cids[k]
            dup = e if dup is None else (dup | e)
        firsts.append(valids[k] if dup is None else (valids[k] & ~dup))

    card = None
    s_cand = None
    hcorr = None
    for k in range(nk):
        fk = firsts[k].astype(jnp.float32)
        card = fk if card is None else card + fk
        sk = cvs[k] * fk
        s_cand = sk if s_cand is None else s_cand + sk
        hk = jnp.where(firsts[k] & (cids[k] < hs), _psi_neg(cvs[k]), 0.0)
        hcorr = hk if hcorr is None else hcorr + hk
    term1 = _psi_neg(-(s_cand / jnp.maximum(card, 1.0)))
    term2 = hsum_ref[...] - hcorr

    colsb = cols_ref[...]
    row_ok = colsb >= 0
    pt = _psi_neg(gt_ref[...])
    iscand = None
    for k in range(nk):
        e = cids[k] == colsb
        iscand = e if iscand is None else (iscand | e)
    t3 = jnp.sum(jnp.where(row_ok & ~iscand, pt, 0.0), axis=0,
                 keepdims=True) * tail_scale

    part = jnp.sum(term1 + term2 + t3, axis=1, keepdims=True) * inv_b

    @pl.when(pl.program_id(0) == 0)
    def _init():
        out_ref[...] = jnp.zeros_like(out_ref)

    out_ref[...] += part


def _tf2x32(k1, k2, x0, x1):
    rot = [[13, 15, 26, 6], [17, 29, 16, 24]]
    ks0, ks1 = np.uint32(k1), np.uint32(k2)
    ks2 = np.uint32(ks0 ^ ks1 ^ np.uint32(0x1BD11BDA))
    x0 = (np.asarray(x0, np.uint32) + ks0).astype(np.uint32)
    x1 = (np.asarray(x1, np.uint32) + ks1).astype(np.uint32)
    ks = [ks1, ks2, ks0]
    for i in range(5):
        for d in rot[i % 2]:
            x0 = (x0 + x1).astype(np.uint32)
            x1 = ((x1 << np.uint32(d)) | (x1 >> np.uint32(32 - d))).astype(
                np.uint32)
            x1 = (x1 ^ x0).astype(np.uint32)
        x0 = (x0 + ks[i % 3]).astype(np.uint32)
        x1 = (x1 + ks[(i + 1) % 3] + np.uint32(i + 1)).astype(np.uint32)
    return x0, x1


def _np_randint(seed, n, minval, maxval):
    def iota_pair(m):
        idx = np.arange(m, dtype=np.uint64)
        return ((idx >> np.uint64(32)).astype(np.uint32),
                (idx & np.uint64(0xFFFFFFFF)).astype(np.uint32))

    def bits32(kk, m):
        c1, c2 = iota_pair(m)
        b1, b2 = _tf2x32(kk[0], kk[1], c1, c2)
        return (b1 ^ b2).astype(np.uint32)

    c1, c2 = iota_pair(2)
    b1, b2 = _tf2x32(np.uint32(seed >> 32), np.uint32(seed & 0xFFFFFFFF),
                     c1, c2)
    hi = bits32((b1[0], b2[0]), n)
    lo = bits32((b1[1], b2[1]), n)
    span = np.uint32(maxval - minval)
    with np.errstate(over="ignore"):
        m = np.uint32(np.uint32(2 ** 16) % span)
        mult = np.uint32(m * m) % span
        off = ((hi % span) * mult + lo % span) % span
    return np.int32(minval) + off.astype(np.int32)


@functools.lru_cache(maxsize=None)
def _constants(b, hs, ts, ss):
    cols = np.asarray(hs + _np_randint(42, ss, 0, ts), np.int32)
    ssq = ss // _NQ
    qr = _NQ * _QROWS
    tidx = np.full((_NQ, _QROWS), cols[0], np.int32)
    tidx[:, :ssq] = cols.reshape(_NQ, ssq)
    pos = (np.arange(ss) // ssq) * _QROWS + np.arange(ss) % ssq
    colsarr = np.full((qr,), -1, np.int32)
    colsarr[pos] = cols
    colsb = np.ascontiguousarray(
        np.broadcast_to(colsarr[:, None], (qr, _LW)))
    return tidx, colsb


def kernel(logits, candidates):
    b, n = logits.shape
    nk = candidates.shape[1]
    hs = min(_HEAD_SIZE, n)
    ts = n - hs
    ss = min(_TAIL_SAMPLE, ts)
    nt = b // _LW
    nw = nt * _NQ
    qr = _NQ * _QROWS

    tidx, colsb = _constants(b, hs, ts, ss)

    lt = logits.T
    c_t = candidates.T.astype(jnp.int32)

    c_safe = jnp.clip(c_t, 0, n - 1)
    arr = c_safe.reshape(nk, nt, _LW).transpose(1, 0, 2)
    cidx = arr.reshape(nw, _CCH, _CHW)

    gt, gc = _make_sc_gather(n, b, nk)(lt, tidx, cidx)

    hsum = pl.pallas_call(
        _head_body,
        grid=(nt,),
        in_specs=[pl.BlockSpec((hs, _LW), lambda i: (0, i))],
        out_specs=pl.BlockSpec((1, _LW), lambda i: (0, i)),
        out_shape=jax.ShapeDtypeStruct((1, b), jnp.float32),
    )(lt)

    out = pl.pallas_call(
        functools.partial(
            _combine_body, nk, hs, float(ts) / float(ss), 1.0 / b),
        grid=(nt,),
        in_specs=[
            pl.BlockSpec((1, _LW), lambda i: (0, i)),
            pl.BlockSpec((qr, _LW), lambda i: (0, i)),
            pl.BlockSpec((b * nk // nt, _LW), lambda i: (i, 0)),
            pl.BlockSpec((nk, _LW), lambda i: (0, i)),
            pl.BlockSpec((qr, _LW), lambda i: (0, 0)),
        ],
        out_specs=pl.BlockSpec((1, 1), lambda i: (0, 0)),
        out_shape=jax.ShapeDtypeStruct((1, 1), jnp.float32),
    )(hsum, gt, gc, c_t, colsb)
    return out[0, 0]


# speedup vs baseline: 56.4544x; 1.0438x over previous
import functools

import numpy as np

import jax
import jax.numpy as jnp
from jax import lax
from jax.experimental import pallas as pl
from jax.experimental.pallas import tpu as pltpu
from jax.experimental.pallas import tpu_sc as plsc

_HEAD_SIZE = 2000
_TAIL_SAMPLE = 100
_CLIP = 20.0
_LW = 128
_NQ = 4
_QROWS = 32


def _psi_neg(x):
    return jnp.log1p(jnp.exp(jnp.minimum(x, _CLIP)))


@functools.lru_cache(maxsize=None)
def _make_sc_gather(n_classes, b, nk):
    info = plsc.get_sparse_core_info()
    nc, ns = info.num_cores, info.num_subcores
    nt = b // _LW
    lpq = _LW // _NQ

    mesh = plsc.VectorSubcoreMesh(core_axis_name="c", subcore_axis_name="s")

    @functools.partial(
        pl.kernel,
        mesh=mesh,
        out_type=[
            jax.ShapeDtypeStruct((_NQ * _QROWS, b), jnp.float32),
            jax.ShapeDtypeStruct((nk * b,), jnp.float32),
        ],
        scratch_types=[
            pltpu.VMEM((_QROWS,), jnp.int32),
            pltpu.VMEM((nk, _LW), jnp.int32),
            pltpu.VMEM((_QROWS, _LW), jnp.float32),
            pltpu.VMEM((nk * lpq, _LW), jnp.float32),
            pltpu.VMEM((nk * lpq,), jnp.float32),
            pltpu.SemaphoreType.DMA,
        ],
    )
    def sc_gather(lt_hbm, ct_hbm, tidx_hbm, gt_hbm, gcv_hbm,
                  tidx_v, cidx_v, tvals_v, cvals_v, ext_v, sem):
        wid = lax.axis_index("s") * nc + lax.axis_index("c")
        j = wid // _NQ
        q = wid % _NQ
        col0 = pl.multiple_of(j * _LW, _LW)
        lane0 = pl.multiple_of(q * lpq, lpq)
        pltpu.sync_copy(tidx_hbm.at[q], tidx_v)
        pltpu.sync_copy(ct_hbm.at[:, pl.ds(col0, _LW)], cidx_v)
        cp_t = pltpu.async_copy(
            lt_hbm.at[tidx_v, pl.ds(col0, _LW)], tvals_v, sem)
        cps = [
            pltpu.async_copy(
                lt_hbm.at[cidx_v.at[k, pl.ds(lane0, lpq)],
                          pl.ds(col0, _LW)],
                cvals_v.at[pl.ds(k * lpq, lpq)], sem)
            for k in range(nk)
        ]
        cp_t.wait()
        pltpu.sync_copy(
            tvals_v, gt_hbm.at[pl.ds(q * _QROWS, _QROWS), pl.ds(col0, _LW)])
        for cp in cps:
            cp.wait()
        lanes = lax.iota(jnp.int32, 16)
        for k in range(nk):
            for h in range(lpq // 16):
                base = k * lpq + h * 16
                coff = lane0 + h * 16
                acc = None
                for i in range(16):
                    v = cvals_v[base + i, pl.ds(coff, 16)]
                    piece = jnp.where(lanes == i, v, 0.0)
                    acc = piece if acc is None else acc + piece
                ext_v[pl.ds(base, 16)] = acc
        for k in range(nk):
            pltpu.sync_copy(
                ext_v.at[pl.ds(k * lpq, lpq)],
                gcv_hbm.at[pl.ds(k * b + j * _LW + q * lpq, lpq)])

    return sc_gather


def _head_body(lt_ref, out_ref):
    out_ref[...] = jnp.sum(_psi_neg(lt_ref[...]), axis=0, keepdims=True)


def _combine_body(nk, hs, tail_scale, inv_b,
                  hsum_ref, gt_ref, gcv_ref, cid_ref, cols_ref, out_ref):
    cids = [cid_ref[k:k + 1, :] for k in range(nk)]
    cvs = [jnp.clip(gcv_ref[k:k + 1, :], -_CLIP, _CLIP) for k in range(nk)]

    valids = [cid >= 0 for cid in cids]
    firsts = []
    for k in range(nk):
        dup = None
        for m in range(k):
            e = cids[m] == cids[k]
            dup = e if dup is None else (dup | e)
        firsts.append(valids[k] if dup is None else (valids[k] & ~dup))

    card = None
    s_cand = None
    hcorr = None
    for k in range(nk):
        fk = firsts[k].astype(jnp.float32)
        card = fk if card is None else card + fk
        sk = cvs[k] * fk
        s_cand = sk if s_cand is None else s_cand + sk
        hk = jnp.where(firsts[k] & (cids[k] < hs), _psi_neg(cvs[k]), 0.0)
        hcorr = hk if hcorr is None else hcorr + hk
    term1 = _psi_neg(-(s_cand / jnp.maximum(card, 1.0)))
    term2 = hsum_ref[...] - hcorr

    colsb = jnp.broadcast_to(cols_ref[...], gt_ref.shape)
    row_ok = colsb >= 0
    pt = _psi_neg(gt_ref[...])
    iscand = None
    for k in range(nk):
        e = cids[k] == colsb
        iscand = e if iscand is None else (iscand | e)
    t3 = jnp.sum(jnp.where(row_ok & ~iscand, pt, 0.0), axis=0,
                 keepdims=True) * tail_scale

    out_ref[...] = jnp.sum(term1 + term2 + t3, axis=1, keepdims=True) * inv_b


def _tf2x32(k1, k2, x0, x1):
    rot = [[13, 15, 26, 6], [17, 29, 16, 24]]
    ks0, ks1 = np.uint32(k1), np.uint32(k2)
    ks2 = np.uint32(ks0 ^ ks1 ^ np.uint32(0x1BD11BDA))
    x0 = (np.asarray(x0, np.uint32) + ks0).astype(np.uint32)
    x1 = (np.asarray(x1, np.uint32) + ks1).astype(np.uint32)
    ks = [ks1, ks2, ks0]
    for i in range(5):
        for d in rot[i % 2]:
            x0 = (x0 + x1).astype(np.uint32)
            x1 = ((x1 << np.uint32(d)) | (x1 >> np.uint32(32 - d))).astype(
                np.uint32)
            x1 = (x1 ^ x0).astype(np.uint32)
        x0 = (x0 + ks[i % 3]).astype(np.uint32)
        x1 = (x1 + ks[(i + 1) % 3] + np.uint32(i + 1)).astype(np.uint32)
    return x0, x1


def _np_randint(seed, n, minval, maxval):
    def iota_pair(m):
        idx = np.arange(m, dtype=np.uint64)
        return ((idx >> np.uint64(32)).astype(np.uint32),
                (idx & np.uint64(0xFFFFFFFF)).astype(np.uint32))

    def bits32(kk, m):
        c1, c2 = iota_pair(m)
        b1, b2 = _tf2x32(kk[0], kk[1], c1, c2)
        return (b1 ^ b2).astype(np.uint32)

    c1, c2 = iota_pair(2)
    b1, b2 = _tf2x32(np.uint32(seed >> 32), np.uint32(seed & 0xFFFFFFFF),
                     c1, c2)
    hi = bits32((b1[0], b2[0]), n)
    lo = bits32((b1[1], b2[1]), n)
    span = np.uint32(maxval - minval)
    with np.errstate(over="ignore"):
        m = np.uint32(np.uint32(2 ** 16) % span)
        mult = np.uint32(m * m) % span
        off = ((hi % span) * mult + lo % span) % span
    return np.int32(minval) + off.astype(np.int32)


@functools.lru_cache(maxsize=None)
def _constants(b, hs, ts, ss):
    cols = np.asarray(hs + _np_randint(42, ss, 0, ts), np.int32)
    ssq = ss // _NQ
    qr = _NQ * _QROWS
    tidx = np.full((_NQ, _QROWS), cols[0], np.int32)
    tidx[:, :ssq] = cols.reshape(_NQ, ssq)
    pos = (np.arange(ss) // ssq) * _QROWS + np.arange(ss) % ssq
    colsarr = np.full((qr,), -1, np.int32)
    colsarr[pos] = cols
    return tidx, colsarr.reshape(qr, 1)


def kernel(logits, candidates):
    b, n = logits.shape
    nk = candidates.shape[1]
    hs = min(_HEAD_SIZE, n)
    ts = n - hs
    ss = min(_TAIL_SAMPLE, ts)
    nt = b // _LW
    qr = _NQ * _QROWS

    tidx, colsc = _constants(b, hs, ts, ss)

    lt = logits.T
    c_t = candidates.T.astype(jnp.int32)

    gt, gcv = _make_sc_gather(n, b, nk)(lt, c_t, tidx)
    gck = gcv.reshape(nk, b)

    hsum = pl.pallas_call(
        _head_body,
        grid=(nt,),
        in_specs=[pl.BlockSpec((hs, _LW), lambda i: (0, i))],
        out_specs=pl.BlockSpec((1, _LW), lambda i: (0, i)),
        out_shape=jax.ShapeDtypeStruct((1, b), jnp.float32),
    )(lt)

    out = pl.pallas_call(
        functools.partial(
            _combine_body, nk, hs, float(ts) / float(ss), 1.0 / b),
        in_specs=[
            pl.BlockSpec((1, b), lambda: (0, 0)),
            pl.BlockSpec((qr, b), lambda: (0, 0)),
            pl.BlockSpec((nk, b), lambda: (0, 0)),
            pl.BlockSpec((nk, b), lambda: (0, 0)),
            pl.BlockSpec((qr, 1), lambda: (0, 0)),
        ],
        out_specs=pl.BlockSpec((1, 1), lambda: (0, 0)),
        out_shape=jax.ShapeDtypeStruct((1, 1), jnp.float32),
    )(hsum, gt, gck, c_t, colsc)
    return out[0, 0]


# speedup vs baseline: 59.0295x; 1.0456x over previous
import functools

import numpy as np

import jax
import jax.numpy as jnp
from jax import lax
from jax.experimental import pallas as pl
from jax.experimental.pallas import tpu as pltpu
from jax.experimental.pallas import tpu_sc as plsc

_HEAD_SIZE = 2000
_TAIL_SAMPLE = 100
_CLIP = 20.0
_LW = 128
_NQ = 4
_QROWS = 32


def _psi_neg(x):
    return jnp.log1p(jnp.exp(jnp.minimum(x, _CLIP)))


@functools.lru_cache(maxsize=None)
def _make_sc_gather(n_classes, b, nk):
    info = plsc.get_sparse_core_info()
    nc, ns = info.num_cores, info.num_subcores
    lpq = _LW // _NQ
    nh = lpq // 16

    mesh = plsc.VectorSubcoreMesh(core_axis_name="c", subcore_axis_name="s")

    @functools.partial(
        pl.kernel,
        mesh=mesh,
        out_type=[
            jax.ShapeDtypeStruct((_NQ * _QROWS, b), jnp.float32),
            jax.ShapeDtypeStruct((nk, b), jnp.float32),
        ],
        scratch_types=[
            pltpu.VMEM((_QROWS,), jnp.int32),
            pltpu.VMEM((nk, _LW), jnp.int32),
            pltpu.VMEM((_QROWS, _LW), jnp.float32),
            pltpu.VMEM((nk * lpq, _LW), jnp.float32),
            pltpu.VMEM((nk * lpq,), jnp.float32),
            pltpu.SemaphoreType.DMA,
        ],
    )
    def sc_gather(lt_hbm, ct_hbm, tidx_hbm, gt_hbm, gcv_hbm,
                  tidx_v, cidx_v, tvals_v, cvals_v, ext_v, sem):
        wid = lax.axis_index("s") * nc + lax.axis_index("c")
        j = wid // _NQ
        q = wid % _NQ
        col0 = pl.multiple_of(j * _LW, _LW)
        lane0 = pl.multiple_of(q * lpq, lpq)
        qrow0 = pl.multiple_of(q * _QROWS, _QROWS)
        pltpu.sync_copy(tidx_hbm.at[pl.ds(qrow0, _QROWS)], tidx_v)
        pltpu.sync_copy(ct_hbm.at[:, pl.ds(col0, _LW)], cidx_v)
        cp_t = pltpu.async_copy(
            lt_hbm.at[tidx_v, pl.ds(col0, _LW)], tvals_v, sem)
        cps = [
            pltpu.async_copy(
                lt_hbm.at[cidx_v.at[k, pl.ds(lane0, lpq)],
                          pl.ds(col0, _LW)],
                cvals_v.at[pl.ds(k * lpq, lpq)], sem)
            for k in range(nk)
        ]
        cp_t.wait()
        pltpu.sync_copy(
            tvals_v, gt_hbm.at[pl.ds(q * _QROWS, _QROWS), pl.ds(col0, _LW)])
        for cp in cps:
            cp.wait()
        lanes = lax.iota(jnp.int32, 16)
        for k in range(nk):
            for h in range(nh):
                base = k * lpq + h * 16
                coff = lane0 + h * 16
                acc = None
                for i in range(16):
                    v = cvals_v[base + i, pl.ds(coff, 16)]
                    piece = jnp.where(lanes == i, v, 0.0)
                    acc = piece if acc is None else acc + piece
                ext_v[pl.ds(base, 16)] = acc
        for k in range(nk):
            pltpu.sync_copy(
                ext_v.at[pl.ds(k * lpq, lpq)],
                gcv_hbm.at[k, pl.ds(col0 + lane0, lpq)])

    return sc_gather


def _head_body(lt_ref, out_ref):
    out_ref[...] = jnp.sum(_psi_neg(lt_ref[...]), axis=0, keepdims=True)


def _combine_body(nk, hs, tail_scale, inv_b,
                  hsum_ref, gt_ref, gcv_ref, cid_ref, cols_ref, out_ref):
    cids = [cid_ref[k:k + 1, :] for k in range(nk)]
    cvs = [jnp.clip(gcv_ref[k:k + 1, :], -_CLIP, _CLIP) for k in range(nk)]

    valids = [cid >= 0 for cid in cids]
    firsts = []
    for k in range(nk):
        dup = None
        for m in range(k):
            e = cids[m] == cids[k]
            dup = e if dup is None else (dup | e)
        firsts.append(valids[k] if dup is None else (valids[k] & ~dup))

    card = None
    s_cand = None
    hcorr = None
    for k in range(nk):
        fk = firsts[k].astype(jnp.float32)
        card = fk if card is None else card + fk
        sk = cvs[k] * fk
        s_cand = sk if s_cand is None else s_cand + sk
        hk = jnp.where(firsts[k] & (cids[k] < hs), _psi_neg(cvs[k]), 0.0)
        hcorr = hk if hcorr is None else hcorr + hk
    term1 = _psi_neg(-(s_cand / jnp.maximum(card, 1.0)))
    term2 = hsum_ref[...] - hcorr

    colsb = jnp.broadcast_to(cols_ref[...], gt_ref.shape)
    row_ok = colsb >= 0
    pt = _psi_neg(gt_ref[...])
    iscand = None
    for k in range(nk):
        e = cids[k] == colsb
        iscand = e if iscand is None else (iscand | e)
    t3 = jnp.sum(jnp.where(row_ok & ~iscand, pt, 0.0), axis=0,
                 keepdims=True) * tail_scale

    out_ref[...] = jnp.sum(term1 + term2 + t3, axis=1, keepdims=True) * inv_b


def _tf2x32(k1, k2, x0, x1):
    rot = [[13, 15, 26, 6], [17, 29, 16, 24]]
    ks0, ks1 = np.uint32(k1), np.uint32(k2)
    ks2 = np.uint32(ks0 ^ ks1 ^ np.uint32(0x1BD11BDA))
    x0 = (np.asarray(x0, np.uint32) + ks0).astype(np.uint32)
    x1 = (np.asarray(x1, np.uint32) + ks1).astype(np.uint32)
    ks = [ks1, ks2, ks0]
    for i in range(5):
        for d in rot[i % 2]:
            x0 = (x0 + x1).astype(np.uint32)
            x1 = ((x1 << np.uint32(d)) | (x1 >> np.uint32(32 - d))).astype(
                np.uint32)
            x1 = (x1 ^ x0).astype(np.uint32)
        x0 = (x0 + ks[i % 3]).astype(np.uint32)
        x1 = (x1 + ks[(i + 1) % 3] + np.uint32(i + 1)).astype(np.uint32)
    return x0, x1


def _np_randint(seed, n, minval, maxval):
    def iota_pair(m):
        idx = np.arange(m, dtype=np.uint64)
        return ((idx >> np.uint64(32)).astype(np.uint32),
                (idx & np.uint64(0xFFFFFFFF)).astype(np.uint32))

    def bits32(kk, m):
        c1, c2 = iota_pair(m)
        b1, b2 = _tf2x32(kk[0], kk[1], c1, c2)
        return (b1 ^ b2).astype(np.uint32)

    c1, c2 = iota_pair(2)
    b1, b2 = _tf2x32(np.uint32(seed >> 32), np.uint32(seed & 0xFFFFFFFF),
                     c1, c2)
    hi = bits32((b1[0], b2[0]), n)
    lo = bits32((b1[1], b2[1]), n)
    span = np.uint32(maxval - minval)
    with np.errstate(over="ignore"):
        m = np.uint32(np.uint32(2 ** 16) % span)
        mult = np.uint32(m * m) % span
        off = ((hi % span) * mult + lo % span) % span
    return np.int32(minval) + off.astype(np.int32)


@functools.lru_cache(maxsize=None)
def _constants(b, hs, ts, ss):
    cols = np.asarray(hs + _np_randint(42, ss, 0, ts), np.int32)
    ssq = ss // _NQ
    qr = _NQ * _QROWS
    tidx = np.full((_NQ, _QROWS), cols[0], np.int32)
    tidx[:, :ssq] = cols.reshape(_NQ, ssq)
    pos = (np.arange(ss) // ssq) * _QROWS + np.arange(ss) % ssq
    colsarr = np.full((qr,), -1, np.int32)
    colsarr[pos] = cols
    return tidx, colsarr.reshape(qr, 1)


def kernel(logits, candidates):
    b, n = logits.shape
    nk = candidates.shape[1]
    hs = min(_HEAD_SIZE, n)
    ts = n - hs
    ss = min(_TAIL_SAMPLE, ts)
    nt = b // _LW
    qr = _NQ * _QROWS

    tidx, colsc = _constants(b, hs, ts, ss)

    lt = logits.T
    c_t = candidates.T.astype(jnp.int32)

    gt, gck = _make_sc_gather(n, b, nk)(lt, c_t, tidx.reshape(-1))

    hsum = pl.pallas_call(
        _head_body,
        grid=(nt,),
        in_specs=[pl.BlockSpec((hs, _LW), lambda i: (0, i))],
        out_specs=pl.BlockSpec((1, _LW), lambda i: (0, i)),
        out_shape=jax.ShapeDtypeStruct((1, b), jnp.float32),
    )(lt)

    out = pl.pallas_call(
        functools.partial(
            _combine_body, nk, hs, float(ts) / float(ss), 1.0 / b),
        in_specs=[
            pl.BlockSpec((1, b), lambda: (0, 0)),
            pl.BlockSpec((qr, b), lambda: (0, 0)),
            pl.BlockSpec((nk, b), lambda: (0, 0)),
            pl.BlockSpec((nk, b), lambda: (0, 0)),
            pl.BlockSpec((qr, 1), lambda: (0, 0)),
        ],
        out_specs=pl.BlockSpec((1, 1), lambda: (0, 0)),
        out_shape=jax.ShapeDtypeStruct((1, 1), jnp.float32),
    )(hsum, gt, gck, c_t, colsc)
    return out[0, 0]


# speedup vs baseline: 60.5214x; 1.0253x over previous
import functools

import numpy as np

import jax
import jax.numpy as jnp
from jax import lax
from jax.experimental import pallas as pl
from jax.experimental.pallas import tpu as pltpu
from jax.experimental.pallas import tpu_sc as plsc

_HEAD_SIZE = 2000
_TAIL_SAMPLE = 100
_CLIP = 20.0
_LW = 128
_NQ = 4
_QROWS = 32


def _psi_neg(x):
    return jnp.log1p(jnp.exp(jnp.minimum(x, _CLIP)))


@functools.lru_cache(maxsize=None)
def _make_sc_gather(n_classes, b, nk):
    info = plsc.get_sparse_core_info()
    nc, ns = info.num_cores, info.num_subcores
    lpq = _LW // _NQ
    nh = lpq // 16

    mesh = plsc.VectorSubcoreMesh(core_axis_name="c", subcore_axis_name="s")

    @functools.partial(
        pl.kernel,
        mesh=mesh,
        out_type=[
            jax.ShapeDtypeStruct((_NQ * _QROWS, b), jnp.float32),
            jax.ShapeDtypeStruct((nk, b), jnp.float32),
        ],
        scratch_types=[
            pltpu.VMEM((_QROWS,), jnp.int32),
            pltpu.VMEM((nk, _LW), jnp.int32),
            pltpu.VMEM((_QROWS, _LW), jnp.float32),
            pltpu.VMEM((nk * lpq, _LW), jnp.float32),
            pltpu.VMEM((nk * lpq,), jnp.float32),
            pltpu.SemaphoreType.DMA,
        ],
    )
    def sc_gather(lt_hbm, ct_hbm, tidx_hbm, gt_hbm, gcv_hbm,
                  tidx_v, cidx_v, tvals_v, cvals_v, ext_v, sem):
        wid = lax.axis_index("s") * nc + lax.axis_index("c")
        j = wid // _NQ
        q = wid % _NQ
        col0 = pl.multiple_of(j * _LW, _LW)
        lane0 = pl.multiple_of(q * lpq, lpq)
        qrow0 = pl.multiple_of(q * _QROWS, _QROWS)
        pltpu.sync_copy(tidx_hbm.at[pl.ds(qrow0, _QROWS)], tidx_v)
        pltpu.sync_copy(ct_hbm.at[:, pl.ds(col0, _LW)], cidx_v)
        cp_t = pltpu.async_copy(
            lt_hbm.at[tidx_v, pl.ds(col0, _LW)], tvals_v, sem)
        cps = [
            pltpu.async_copy(
                lt_hbm.at[cidx_v.at[k, pl.ds(lane0, lpq)],
                          pl.ds(col0, _LW)],
                cvals_v.at[pl.ds(k * lpq, lpq)], sem)
            for k in range(nk)
        ]
        cp_t.wait()
        pltpu.sync_copy(
            tvals_v, gt_hbm.at[pl.ds(q * _QROWS, _QROWS), pl.ds(col0, _LW)])
        lanes = lax.iota(jnp.int32, 16)
        for k in range(nk):
            cps[k].wait()
            for h in range(nh):
                base = k * lpq + h * 16
                coff = lane0 + h * 16
                acc = None
                for i in range(16):
                    v = cvals_v[base + i, pl.ds(coff, 16)]
                    piece = jnp.where(lanes == i, v, 0.0)
                    acc = piece if acc is None else acc + piece
                ext_v[pl.ds(base, 16)] = acc
        for k in range(nk):
            pltpu.sync_copy(
                ext_v.at[pl.ds(k * lpq, lpq)],
                gcv_hbm.at[k, pl.ds(col0 + lane0, lpq)])

    return sc_gather


def _head_body(nk, lt_ref, cid_ref, cols_ref,
               hsum_ref, fm_ref, card_ref, tw_ref):
    hsum_ref[...] = jnp.sum(_psi_neg(lt_ref[...]), axis=0, keepdims=True)

    cids = [cid_ref[k:k + 1, :] for k in range(nk)]
    card = None
    for k in range(nk):
        dup = None
        for m in range(k):
            e = cids[m] == cids[k]
            dup = e if dup is None else (dup | e)
        valid = cids[k] >= 0
        first = valid if dup is None else (valid & ~dup)
        fk = first.astype(jnp.float32)
        fm_ref[k:k + 1, :] = fk
        card = fk if card is None else card + fk
    card_ref[...] = card

    colsb = jnp.broadcast_to(cols_ref[...], tw_ref.shape)
    iscand = None
    for k in range(nk):
        e = cids[k] == colsb
        iscand = e if iscand is None else (iscand | e)
    tw_ref[...] = jnp.where((colsb >= 0) & ~iscand, 1.0, 0.0)


def _combine_body(nk, hs, tail_scale, inv_b,
                  hsum_ref, gt_ref, gcv_ref, cid_ref, fm_ref, card_ref,
                  tw_ref, out_ref):
    cids = [cid_ref[k:k + 1, :] for k in range(nk)]
    cvs = [jnp.clip(gcv_ref[k:k + 1, :], -_CLIP, _CLIP) for k in range(nk)]

    s_cand = None
    hcorr = None
    for k in range(nk):
        fk = fm_ref[k:k + 1, :]
        sk = cvs[k] * fk
        s_cand = sk if s_cand is None else s_cand + sk
        hk = jnp.where((fk > 0.0) & (cids[k] < hs), _psi_neg(cvs[k]), 0.0)
        hcorr = hk if hcorr is None else hcorr + hk
    term1 = _psi_neg(-(s_cand / jnp.maximum(card_ref[...], 1.0)))
    term2 = hsum_ref[...] - hcorr

    pt = _psi_neg(gt_ref[...]) * tw_ref[...]
    t3 = jnp.sum(pt, axis=0, keepdims=True) * tail_scale

    out_ref[...] = jnp.sum(term1 + term2 + t3, axis=1, keepdims=True) * inv_b


def _tf2x32(k1, k2, x0, x1):
    rot = [[13, 15, 26, 6], [17, 29, 16, 24]]
    ks0, ks1 = np.uint32(k1), np.uint32(k2)
    ks2 = np.uint32(ks0 ^ ks1 ^ np.uint32(0x1BD11BDA))
    x0 = (np.asarray(x0, np.uint32) + ks0).astype(np.uint32)
    x1 = (np.asarray(x1, np.uint32) + ks1).astype(np.uint32)
    ks = [ks1, ks2, ks0]
    for i in range(5):
        for d in rot[i % 2]:
            x0 = (x0 + x1).astype(np.uint32)
            x1 = ((x1 << np.uint32(d)) | (x1 >> np.uint32(32 - d))).astype(
                np.uint32)
            x1 = (x1 ^ x0).astype(np.uint32)
        x0 = (x0 + ks[i % 3]).astype(np.uint32)
        x1 = (x1 + ks[(i + 1) % 3] + np.uint32(i + 1)).astype(np.uint32)
    return x0, x1


def _np_randint(seed, n, minval, maxval):
    def iota_pair(m):
        idx = np.arange(m, dtype=np.uint64)
        return ((idx >> np.uint64(32)).astype(np.uint32),
                (idx & np.uint64(0xFFFFFFFF)).astype(np.uint32))

    def bits32(kk, m):
        c1, c2 = iota_pair(m)
        b1, b2 = _tf2x32(kk[0], kk[1], c1, c2)
        return (b1 ^ b2).astype(np.uint32)

    c1, c2 = iota_pair(2)
    b1, b2 = _tf2x32(np.uint32(seed >> 32), np.uint32(seed & 0xFFFFFFFF),
                     c1, c2)
    hi = bits32((b1[0], b2[0]), n)
    lo = bits32((b1[1], b2[1]), n)
    span = np.uint32(maxval - minval)
    with np.errstate(over="ignore"):
        m = np.uint32(np.uint32(2 ** 16) % span)
        mult = np.uint32(m * m) % span
        off = ((hi % span) * mult + lo % span) % span
    return np.int32(minval) + off.astype(np.int32)


@functools.lru_cache(maxsize=None)
def _constants(b, hs, ts, ss):
    cols = np.asarray(hs + _np_randint(42, ss, 0, ts), np.int32)
    ssq = ss // _NQ
    qr = _NQ * _QROWS
    tidx = np.full((_NQ, _QROWS), cols[0], np.int32)
    tidx[:, :ssq] = cols.reshape(_NQ, ssq)
    pos = (np.arange(ss) // ssq) * _QROWS + np.arange(ss) % ssq
    colsarr = np.full((qr,), -1, np.int32)
    colsarr[pos] = cols
    return tidx, colsarr.reshape(qr, 1)


def kernel(logits, candidates):
    b, n = logits.shape
    nk = candidates.shape[1]
    hs = min(_HEAD_SIZE, n)
    ts = n - hs
    ss = min(_TAIL_SAMPLE, ts)
    nt = b // _LW
    qr = _NQ * _QROWS

    tidx, colsc = _constants(b, hs, ts, ss)

    lt = logits.T
    c_t = candidates.T.astype(jnp.int32)

    gt, gck = _make_sc_gather(n, b, nk)(lt, c_t, tidx.reshape(-1))

    hsum, fm, card, tw = pl.pallas_call(
        functools.partial(_head_body, nk),
        grid=(nt,),
        in_specs=[
            pl.BlockSpec((hs, _LW), lambda i: (0, i)),
            pl.BlockSpec((nk, _LW), lambda i: (0, i)),
            pl.BlockSpec((qr, 1), lambda i: (0, 0)),
        ],
        out_specs=[
            pl.BlockSpec((1, _LW), lambda i: (0, i)),
            pl.BlockSpec((nk, _LW), lambda i: (0, i)),
            pl.BlockSpec((1, _LW), lambda i: (0, i)),
            pl.BlockSpec((qr, _LW), lambda i: (0, i)),
        ],
        out_shape=[
            jax.ShapeDtypeStruct((1, b), jnp.float32),
            jax.ShapeDtypeStruct((nk, b), jnp.float32),
            jax.ShapeDtypeStruct((1, b), jnp.float32),
            jax.ShapeDtypeStruct((qr, b), jnp.float32),
        ],
    )(lt, c_t, colsc)

    out = pl.pallas_call(
        functools.partial(
            _combine_body, nk, hs, float(ts) / float(ss), 1.0 / b),
        in_specs=[
            pl.BlockSpec((1, b), lambda: (0, 0)),
            pl.BlockSpec((qr, b), lambda: (0, 0)),
            pl.BlockSpec((nk, b), lambda: (0, 0)),
            pl.BlockSpec((nk, b), lambda: (0, 0)),
            pl.BlockSpec((nk, b), lambda: (0, 0)),
            pl.BlockSpec((1, b), lambda: (0, 0)),
            pl.BlockSpec((qr, b), lambda: (0, 0)),
        ],
        out_specs=pl.BlockSpec((1, 1), lambda: (0, 0)),
        out_shape=jax.ShapeDtypeStruct((1, 1), jnp.float32),
    )(hsum, gt, gck, c_t, fm, card, tw)
    return out[0, 0]
